# baseline (speedup 1.0000x reference)
import jax
import jax.numpy as jnp
from jax.experimental import pallas as pl

BATCH = 16384
MF_DIM = 16
MLP_DIM = 32


def _mlp_body(umfT_ref, mmfT_ref, umlpT_ref, mmlpT_ref,
              w1a_ref, w1b_ref, b1_ref, w2_ref, b2_ref, w3_ref, b3_ref,
              wfa_ref, wfb_ref, bf_ref, out_ref):
    h1 = jnp.maximum(
        jnp.dot(w1a_ref[...], umlpT_ref[...], preferred_element_type=jnp.float32)
        + jnp.dot(w1b_ref[...], mmlpT_ref[...], preferred_element_type=jnp.float32)
        + b1_ref[...], 0.0)
    h2 = jnp.maximum(
        jnp.dot(w2_ref[...], h1, preferred_element_type=jnp.float32)
        + b2_ref[...], 0.0)
    h3 = jnp.maximum(
        jnp.dot(w3_ref[...], h2, preferred_element_type=jnp.float32)
        + b3_ref[...], 0.0)
    gmf = umfT_ref[...] * mmfT_ref[...]
    dn = (((0,), (0,)), ((), ()))
    fin = (jax.lax.dot_general(gmf, wfa_ref[...], dn,
                               preferred_element_type=jnp.float32)
           + jax.lax.dot_general(h3, wfb_ref[...], dn,
                                 preferred_element_type=jnp.float32)
           + bf_ref[0, 0])
    out_ref[...] = 5.0 * jax.nn.sigmoid(fin)


def _tc_mlp(umfT, mmfT, umlpT, mmlpT, w1a, w1b, b1, w2, b2, w3, b3,
            wfa, wfb, bf):
    bk = 4096
    grid = (BATCH // bk,)
    full = lambda i: (0, 0)
    col = lambda i: (0, i)
    row = lambda i: (i, 0)
    return pl.pallas_call(
        _mlp_body,
        grid=grid,
        in_specs=[
            pl.BlockSpec((MF_DIM, bk), col),
            pl.BlockSpec((MF_DIM, bk), col),
            pl.BlockSpec((MLP_DIM, bk), col),
            pl.BlockSpec((MLP_DIM, bk), col),
            pl.BlockSpec((2 * MLP_DIM, MLP_DIM), full),
            pl.BlockSpec((2 * MLP_DIM, MLP_DIM), full),
            pl.BlockSpec((2 * MLP_DIM, 1), full),
            pl.BlockSpec((2 * MLP_DIM, 2 * MLP_DIM), full),
            pl.BlockSpec((2 * MLP_DIM, 1), full),
            pl.BlockSpec((MLP_DIM, 2 * MLP_DIM), full),
            pl.BlockSpec((MLP_DIM, 1), full),
            pl.BlockSpec((MF_DIM, 1), full),
            pl.BlockSpec((MLP_DIM, 1), full),
            pl.BlockSpec((1, 1), full),
        ],
        out_specs=pl.BlockSpec((bk, 1), row),
        out_shape=jax.ShapeDtypeStruct((BATCH, 1), jnp.float32),
    )(umfT, mmfT, umlpT, mmlpT, w1a, w1b, b1, w2, b2, w3, b3, wfa, wfb, bf)


def kernel(X, user_mf, movie_mf, user_mlp, movie_mlp,
           W1, b1, W2, b2, W3, b3, Wf, bf):
    uidx = X[:, 0]
    midx = X[:, 1]
    umfT = user_mf.at[uidx].get(mode="promise_in_bounds").T
    mmfT = movie_mf.at[midx].get(mode="promise_in_bounds").T
    umlpT = user_mlp.at[uidx].get(mode="promise_in_bounds").T
    mmlpT = movie_mlp.at[midx].get(mode="promise_in_bounds").T
    w1a = W1[:, :MLP_DIM]
    w1b = W1[:, MLP_DIM:]
    wfa = Wf[:, :MF_DIM].T
    wfb = Wf[:, MF_DIM:].T
    return _tc_mlp(umfT, mmfT, umlpT, mmlpT,
                   w1a, w1b, b1.reshape(-1, 1), W2, b2.reshape(-1, 1),
                   W3, b3.reshape(-1, 1), wfa, wfb, bf.reshape(1, 1))


# speedup vs baseline: 1.1032x; 1.1032x over previous
import jax
import jax.numpy as jnp
from jax.experimental import pallas as pl

BATCH = 16384
MF_DIM = 16
MLP_DIM = 32


def _mlp_body(umfT_ref, mmfT_ref, umlpT_ref, mmlpT_ref,
              w1a_ref, w1b_ref, b1_ref, w2_ref, b2_ref, w3_ref, b3_ref,
              wfa_ref, wfb_ref, bf_ref, out_ref):
    h1 = jnp.maximum(
        jnp.dot(w1a_ref[...], umlpT_ref[...], preferred_element_type=jnp.float32)
        + jnp.dot(w1b_ref[...], mmlpT_ref[...], preferred_element_type=jnp.float32)
        + b1_ref[...], 0.0)
    h2 = jnp.maximum(
        jnp.dot(w2_ref[...], h1, preferred_element_type=jnp.float32)
        + b2_ref[...], 0.0)
    h3 = jnp.maximum(
        jnp.dot(w3_ref[...], h2, preferred_element_type=jnp.float32)
        + b3_ref[...], 0.0)
    gmf = umfT_ref[...] * mmfT_ref[...]
    fin = (jnp.dot(wfa_ref[...], gmf, preferred_element_type=jnp.float32)
           + jnp.dot(wfb_ref[...], h3, preferred_element_type=jnp.float32)
           + bf_ref[0, 0])
    out_ref[...] = 5.0 * jax.nn.sigmoid(fin)


def _tc_mlp(umfT, mmfT, umlpT, mmlpT, w1a, w1b, b1, w2, b2, w3, b3,
            wfa, wfb, bf):
    bk = 4096
    grid = (BATCH // bk,)
    full = lambda i: (0, 0)
    col = lambda i: (0, i)
    row = lambda i: (i, 0)
    return pl.pallas_call(
        _mlp_body,
        grid=grid,
        in_specs=[
            pl.BlockSpec((MF_DIM, bk), col),
            pl.BlockSpec((MF_DIM, bk), col),
            pl.BlockSpec((MLP_DIM, bk), col),
            pl.BlockSpec((MLP_DIM, bk), col),
            pl.BlockSpec((2 * MLP_DIM, MLP_DIM), full),
            pl.BlockSpec((2 * MLP_DIM, MLP_DIM), full),
            pl.BlockSpec((2 * MLP_DIM, 1), full),
            pl.BlockSpec((2 * MLP_DIM, 2 * MLP_DIM), full),
            pl.BlockSpec((2 * MLP_DIM, 1), full),
            pl.BlockSpec((MLP_DIM, 2 * MLP_DIM), full),
            pl.BlockSpec((MLP_DIM, 1), full),
            pl.BlockSpec((1, MF_DIM), full),
            pl.BlockSpec((1, MLP_DIM), full),
            pl.BlockSpec((1, 1), full),
        ],
        out_specs=pl.BlockSpec((1, bk), col),
        out_shape=jax.ShapeDtypeStruct((1, BATCH), jnp.float32),
    )(umfT, mmfT, umlpT, mmlpT, w1a, w1b, b1, w2, b2, w3, b3, wfa, wfb, bf)


def kernel(X, user_mf, movie_mf, user_mlp, movie_mlp,
           W1, b1, W2, b2, W3, b3, Wf, bf):
    uidx = X[:, 0]
    midx = X[:, 1]
    umfT = user_mf.at[uidx].get(mode="promise_in_bounds").T
    mmfT = movie_mf.at[midx].get(mode="promise_in_bounds").T
    umlpT = user_mlp.at[uidx].get(mode="promise_in_bounds").T
    mmlpT = movie_mlp.at[midx].get(mode="promise_in_bounds").T
    w1a = W1[:, :MLP_DIM]
    w1b = W1[:, MLP_DIM:]
    wfa = Wf[:, :MF_DIM]
    wfb = Wf[:, MF_DIM:]
    out = _tc_mlp(umfT, mmfT, umlpT, mmlpT,
                  w1a, w1b, b1.reshape(-1, 1), W2, b2.reshape(-1, 1),
                  W3, b3.reshape(-1, 1), wfa, wfb, bf.reshape(1, 1))
    return out.reshape(BATCH, 1)


# speedup vs baseline: 1.1248x; 1.0195x over previous
import jax
import jax.numpy as jnp
from jax.experimental import pallas as pl

BATCH = 16384
MF_DIM = 16
MLP_DIM = 32


def _mlp_chain_body(umlpT_ref, mmlpT_ref, w1a_ref, w1b_ref, b1_ref,
                    w2_ref, b2_ref, w3_ref, b3_ref, h3_ref):
    h1 = jnp.maximum(
        jnp.dot(w1a_ref[...], umlpT_ref[...], preferred_element_type=jnp.float32)
        + jnp.dot(w1b_ref[...], mmlpT_ref[...], preferred_element_type=jnp.float32)
        + b1_ref[...], 0.0)
    h2 = jnp.maximum(
        jnp.dot(w2_ref[...], h1, preferred_element_type=jnp.float32)
        + b2_ref[...], 0.0)
    h3_ref[...] = jnp.maximum(
        jnp.dot(w3_ref[...], h2, preferred_element_type=jnp.float32)
        + b3_ref[...], 0.0)


def _final_body(umfT_ref, mmfT_ref, h3_ref, wfa_ref, wfb_ref, bf_ref,
                out_ref):
    gmf = umfT_ref[...] * mmfT_ref[...]
    fin = (jnp.dot(wfa_ref[...], gmf, preferred_element_type=jnp.float32)
           + jnp.dot(wfb_ref[...], h3_ref[...], preferred_element_type=jnp.float32)
           + bf_ref[0, 0])
    out_ref[...] = 5.0 * jax.nn.sigmoid(fin)


def _tc_mlp_chain(umlpT, mmlpT, w1a, w1b, b1, w2, b2, w3, b3):
    bk = 4096
    grid = (BATCH // bk,)
    full = lambda i: (0, 0)
    col = lambda i: (0, i)
    return pl.pallas_call(
        _mlp_chain_body,
        grid=grid,
        in_specs=[
            pl.BlockSpec((MLP_DIM, bk), col),
            pl.BlockSpec((MLP_DIM, bk), col),
            pl.BlockSpec((2 * MLP_DIM, MLP_DIM), full),
            pl.BlockSpec((2 * MLP_DIM, MLP_DIM), full),
            pl.BlockSpec((2 * MLP_DIM, 1), full),
            pl.BlockSpec((2 * MLP_DIM, 2 * MLP_DIM), full),
            pl.BlockSpec((2 * MLP_DIM, 1), full),
            pl.BlockSpec((MLP_DIM, 2 * MLP_DIM), full),
            pl.BlockSpec((MLP_DIM, 1), full),
        ],
        out_specs=pl.BlockSpec((MLP_DIM, bk), col),
        out_shape=jax.ShapeDtypeStruct((MLP_DIM, BATCH), jnp.float32),
    )(umlpT, mmlpT, w1a, w1b, b1, w2, b2, w3, b3)


def _tc_final(umfT, mmfT, h3T, wfa, wfb, bf):
    bk = 4096
    grid = (BATCH // bk,)
    full = lambda i: (0, 0)
    col = lambda i: (0, i)
    return pl.pallas_call(
        _final_body,
        grid=grid,
        in_specs=[
            pl.BlockSpec((MF_DIM, bk), col),
            pl.BlockSpec((MF_DIM, bk), col),
            pl.BlockSpec((MLP_DIM, bk), col),
            pl.BlockSpec((1, MF_DIM), full),
            pl.BlockSpec((1, MLP_DIM), full),
            pl.BlockSpec((1, 1), full),
        ],
        out_specs=pl.BlockSpec((1, bk), col),
        out_shape=jax.ShapeDtypeStruct((1, BATCH), jnp.float32),
    )(umfT, mmfT, h3T, wfa, wfb, bf)


def kernel(X, user_mf, movie_mf, user_mlp, movie_mlp,
           W1, b1, W2, b2, W3, b3, Wf, bf):
    uidx = X[:, 0]
    midx = X[:, 1]
    umlpT = user_mlp.at[uidx].get(mode="promise_in_bounds").T
    mmlpT = movie_mlp.at[midx].get(mode="promise_in_bounds").T
    umfT = user_mf.at[uidx].get(mode="promise_in_bounds").T
    mmfT = movie_mf.at[midx].get(mode="promise_in_bounds").T
    w1a = W1[:, :MLP_DIM]
    w1b = W1[:, MLP_DIM:]
    wfa = Wf[:, :MF_DIM]
    wfb = Wf[:, MF_DIM:]
    h3T = _tc_mlp_chain(umlpT, mmlpT, w1a, w1b, b1.reshape(-1, 1),
                        W2, b2.reshape(-1, 1), W3, b3.reshape(-1, 1))
    out = _tc_final(umfT, mmfT, h3T, wfa, wfb, bf.reshape(1, 1))
    return out.reshape(BATCH, 1)
